# baseline (speedup 1.0000x reference)
import functools

import jax
import jax.numpy as jnp
from jax import lax
from jax.experimental import pallas as pl
from jax.experimental.pallas import tpu as pltpu
from jax.experimental.pallas import tpu_sc as plsc

BN = 256
N2 = 1024
C2 = 256
C1 = 128
BIG_I = 1 << 30
INF = 3e38

B = 8
N1 = 4096
NB = N1 // BN
NTOT = B * N1
NW = 32
ROWS_PER_W = NTOT // NW
G = 32
NCHUNK = ROWS_PER_W // G


def _knn_kernel(xyz1_ref, xyz2t_ref, i1_ref, i2_ref, i3_ref,
                w1_ref, w2_ref, w3_ref):
    b = pl.program_id(0)
    x1 = xyz1_ref[0]
    x2t = xyz2t_ref[0]
    sq1 = jnp.sum(x1 * x1, axis=1, keepdims=True)
    sq2 = jnp.sum(x2t * x2t, axis=0, keepdims=True)
    dot = lax.dot_general(x1, x2t, (((1,), (0,)), ((), ())),
                          preferred_element_type=jnp.float32)
    d = sq1 + sq2 - 2.0 * dot

    cidx = lax.broadcasted_iota(jnp.int32, (BN, N2), 1)
    m1 = jnp.min(d, axis=1, keepdims=True)
    i1 = jnp.min(jnp.where(d == m1, cidx, BIG_I), axis=1, keepdims=True)
    e = jnp.where(cidx == i1, INF, d)
    m2 = jnp.min(e, axis=1, keepdims=True)
    i2 = jnp.min(jnp.where(e == m2, cidx, BIG_I), axis=1, keepdims=True)
    f = jnp.where(cidx == i2, INF, e)
    m3 = jnp.min(f, axis=1, keepdims=True)
    i3 = jnp.min(jnp.where(f == m3, cidx, BIG_I), axis=1, keepdims=True)

    r1 = 1.0 / jnp.maximum(m1, 1e-10)
    r2 = 1.0 / jnp.maximum(m2, 1e-10)
    r3 = 1.0 / jnp.maximum(m3, 1e-10)
    norm = r1 + r2 + r3
    base = b * N2
    i1_ref[0] = (i1 + base).reshape(1, BN)
    i2_ref[0] = (i2 + base).reshape(1, BN)
    i3_ref[0] = (i3 + base).reshape(1, BN)
    w1_ref[0] = (r1 / norm).reshape(1, BN)
    w2_ref[0] = (r2 / norm).reshape(1, BN)
    w3_ref[0] = (r3 / norm).reshape(1, BN)


def _table_kernel(p2_ref, w1a_ref, t_ref):
    t_ref[...] = jnp.dot(p2_ref[...], w1a_ref[...],
                         preferred_element_type=jnp.float32)


def _mlp_kernel(x_ref, p1_ref, w1b_ref, w2_ref, b1_ref, b2_ref, out_ref):
    h = x_ref[...] + p1_ref[...] @ w1b_ref[...] + b1_ref[...]
    h = jnp.maximum(h, 0.0)
    o = h @ w2_ref[...] + b2_ref[...]
    out_ref[...] = jnp.maximum(o, 0.0)


def _interp_sc(t_hbm, i1_hbm, i2_hbm, i3_hbm, wa_hbm, wb_hbm, wc_hbm,
               out_hbm, idx1_v, idx2_v, idx3_v, wa_v, wb_v, wc_v,
               rows1_v, rows2_v, rows3_v, out_v, sem):
    wid = lax.axis_index("s") * 2 + lax.axis_index("c")

    def chunk(g, _):
        base = wid * ROWS_PER_W + g * G
        pltpu.sync_copy(i1_hbm.at[pl.ds(base, G)], idx1_v)
        pltpu.sync_copy(i2_hbm.at[pl.ds(base, G)], idx2_v)
        pltpu.sync_copy(i3_hbm.at[pl.ds(base, G)], idx3_v)
        pltpu.sync_copy(wa_hbm.at[pl.ds(base, G)], wa_v)
        pltpu.sync_copy(wb_hbm.at[pl.ds(base, G)], wb_v)
        pltpu.sync_copy(wc_hbm.at[pl.ds(base, G)], wc_v)
        c1 = pltpu.async_copy(t_hbm.at[idx1_v], rows1_v, sem)
        c2 = pltpu.async_copy(t_hbm.at[idx2_v], rows2_v, sem)
        c3 = pltpu.async_copy(t_hbm.at[idx3_v], rows3_v, sem)
        c1.wait()
        c2.wait()
        c3.wait()

        dnums = lax.GatherDimensionNumbers(
            offset_dims=(), collapsed_slice_dims=(0,), start_index_map=(0,))

        def bcast(v16, lane):
            idx = jnp.full((16, 1), lane, jnp.int32)
            return lax.gather(v16, idx, dnums, (1,),
                              mode=lax.GatherScatterMode.PROMISE_IN_BOUNDS)

        def row16(q, _):
            wa16 = wa_v[pl.ds(q * 16, 16)]
            wb16 = wb_v[pl.ds(q * 16, 16)]
            wc16 = wc_v[pl.ds(q * 16, 16)]
            for ri in range(16):
                r = q * 16 + ri
                wa = bcast(wa16, ri)
                wb = bcast(wb16, ri)
                wc = bcast(wc16, ri)
                for j in range(C2 // 16):
                    sl = pl.ds(j * 16, 16)
                    acc = (wa * rows1_v[r, sl] + wb * rows2_v[r, sl]
                           + wc * rows3_v[r, sl])
                    out_v[r, sl] = acc
            return 0

        lax.fori_loop(0, G // 16, row16, 0)
        pltpu.sync_copy(out_v, out_hbm.at[pl.ds(base, G)])
        return 0

    lax.fori_loop(0, NCHUNK, chunk, 0)


_interp_call = pl.kernel(
    _interp_sc,
    out_type=jax.ShapeDtypeStruct((NTOT, C2), jnp.float32),
    mesh=plsc.VectorSubcoreMesh(core_axis_name="c", subcore_axis_name="s"),
    scratch_types=[
        pltpu.VMEM((G,), jnp.int32),
        pltpu.VMEM((G,), jnp.int32),
        pltpu.VMEM((G,), jnp.int32),
        pltpu.VMEM((G,), jnp.float32),
        pltpu.VMEM((G,), jnp.float32),
        pltpu.VMEM((G,), jnp.float32),
        pltpu.VMEM((G, C2), jnp.float32),
        pltpu.VMEM((G, C2), jnp.float32),
        pltpu.VMEM((G, C2), jnp.float32),
        pltpu.VMEM((G, C2), jnp.float32),
        pltpu.SemaphoreType.DMA,
    ],
)


@jax.jit
def kernel(xyz1, xyz2, points1, points2, W1, b1, W2, b2):
    xyz1p = jnp.pad(xyz1, ((0, 0), (0, 0), (0, 5)))
    xyz2t = jnp.pad(xyz2, ((0, 0), (0, 0), (0, 5))).transpose(0, 2, 1)
    w1a = W1[:C2]
    w1b = W1[C2:]

    grid = (B, NB)
    shp_i = jax.ShapeDtypeStruct((B * NB, 1, BN), jnp.int32)
    shp_f = jax.ShapeDtypeStruct((B * NB, 1, BN), jnp.float32)
    out_spec = pl.BlockSpec((1, 1, BN), lambda b, n: (b * NB + n, 0, 0))
    i1, i2, i3, wa, wb, wc = pl.pallas_call(
        _knn_kernel,
        grid=grid,
        in_specs=[
            pl.BlockSpec((1, BN, 8), lambda b, n: (b, n, 0)),
            pl.BlockSpec((1, 8, N2), lambda b, n: (b, 0, 0)),
        ],
        out_specs=[out_spec] * 6,
        out_shape=[shp_i, shp_i, shp_i, shp_f, shp_f, shp_f],
    )(xyz1p, xyz2t)

    p2f = points2.reshape(B * N2, C2)
    table = pl.pallas_call(
        _table_kernel,
        grid=(B * N2 // 512,),
        in_specs=[
            pl.BlockSpec((512, C2), lambda i: (i, 0)),
            pl.BlockSpec((C2, C2), lambda i: (0, 0)),
        ],
        out_specs=pl.BlockSpec((512, C2), lambda i: (i, 0)),
        out_shape=jax.ShapeDtypeStruct((B * N2, C2), jnp.float32),
    )(p2f, w1a)

    interp = _interp_call(
        table,
        i1.reshape(NTOT), i2.reshape(NTOT), i3.reshape(NTOT),
        wa.reshape(NTOT), wb.reshape(NTOT), wc.reshape(NTOT))

    p1f = points1.reshape(NTOT, C1)
    out = pl.pallas_call(
        _mlp_kernel,
        grid=(NTOT // BN,),
        in_specs=[
            pl.BlockSpec((BN, C2), lambda i: (i, 0)),
            pl.BlockSpec((BN, C1), lambda i: (i, 0)),
            pl.BlockSpec((C1, C2), lambda i: (0, 0)),
            pl.BlockSpec((C2, C2), lambda i: (0, 0)),
            pl.BlockSpec((1, C2), lambda i: (0, 0)),
            pl.BlockSpec((1, C2), lambda i: (0, 0)),
        ],
        out_specs=pl.BlockSpec((BN, C2), lambda i: (i, 0)),
        out_shape=jax.ShapeDtypeStruct((NTOT, C2), jnp.float32),
    )(interp, p1f, w1b, W2, b1.reshape(1, C2), b2.reshape(1, C2))
    return out.reshape(B, N1, C2)


# speedup vs baseline: 1.0461x; 1.0461x over previous
import functools

import jax
import jax.numpy as jnp
from jax import lax
from jax.experimental import pallas as pl
from jax.experimental.pallas import tpu as pltpu
from jax.experimental.pallas import tpu_sc as plsc

BN = 256
N2 = 1024
C2 = 256
C1 = 128
BIG_I = 1 << 30
INF = 3e38

B = 8
N1 = 4096
NB = N1 // BN
NTOT = B * N1
NW = 32
ROWS_PER_W = NTOT // NW
G = 32
NCHUNK = ROWS_PER_W // G


def _knn_kernel(xyz1_ref, xyz2t_ref, i1_ref, i2_ref, i3_ref,
                w1_ref, w2_ref, w3_ref):
    b = pl.program_id(0)
    x1 = xyz1_ref[0]
    x2t = xyz2t_ref[0]
    sq1 = jnp.sum(x1 * x1, axis=1, keepdims=True)
    sq2 = jnp.sum(x2t * x2t, axis=0, keepdims=True)
    dot = lax.dot_general(x1, x2t, (((1,), (0,)), ((), ())),
                          preferred_element_type=jnp.float32)
    d = sq1 + sq2 - 2.0 * dot

    cidx = lax.broadcasted_iota(jnp.int32, (BN, N2), 1)
    m1 = jnp.min(d, axis=1, keepdims=True)
    i1 = jnp.min(jnp.where(d == m1, cidx, BIG_I), axis=1, keepdims=True)
    e = jnp.where(cidx == i1, INF, d)
    m2 = jnp.min(e, axis=1, keepdims=True)
    i2 = jnp.min(jnp.where(e == m2, cidx, BIG_I), axis=1, keepdims=True)
    f = jnp.where(cidx == i2, INF, e)
    m3 = jnp.min(f, axis=1, keepdims=True)
    i3 = jnp.min(jnp.where(f == m3, cidx, BIG_I), axis=1, keepdims=True)

    r1 = 1.0 / jnp.maximum(m1, 1e-10)
    r2 = 1.0 / jnp.maximum(m2, 1e-10)
    r3 = 1.0 / jnp.maximum(m3, 1e-10)
    norm = r1 + r2 + r3
    base = b * N2
    i1_ref[0] = (i1 + base).reshape(1, BN)
    i2_ref[0] = (i2 + base).reshape(1, BN)
    i3_ref[0] = (i3 + base).reshape(1, BN)
    w1_ref[0] = (r1 / norm).reshape(1, BN)
    w2_ref[0] = (r2 / norm).reshape(1, BN)
    w3_ref[0] = (r3 / norm).reshape(1, BN)


def _table_kernel(p2_ref, w1a_ref, t_ref):
    t_ref[...] = jnp.dot(p2_ref[...], w1a_ref[...],
                         preferred_element_type=jnp.float32)


def _mlp_kernel(x_ref, p1_ref, w1b_ref, w2_ref, b1_ref, b2_ref, out_ref):
    h = x_ref[...] + p1_ref[...] @ w1b_ref[...] + b1_ref[...]
    h = jnp.maximum(h, 0.0)
    o = h @ w2_ref[...] + b2_ref[...]
    out_ref[...] = jnp.maximum(o, 0.0)


def _interp_sc(t_hbm, i1_hbm, i2_hbm, i3_hbm, wa_hbm, wb_hbm, wc_hbm,
               out_hbm, idx1_v, idx2_v, idx3_v, wa_v, wb_v, wc_v,
               ra1, ra2, ra3, rb1, rb2, rb3, out_v,
               sem_a, sem_b, sem_o):
    wid = lax.axis_index("s") * 2 + lax.axis_index("c")
    row0 = wid * ROWS_PER_W
    crow0 = wid * NCHUNK

    pltpu.sync_copy(i1_hbm.at[pl.ds(crow0, NCHUNK)], idx1_v)
    pltpu.sync_copy(i2_hbm.at[pl.ds(crow0, NCHUNK)], idx2_v)
    pltpu.sync_copy(i3_hbm.at[pl.ds(crow0, NCHUNK)], idx3_v)
    pltpu.sync_copy(wa_hbm.at[pl.ds(crow0, NCHUNK)], wa_v)
    pltpu.sync_copy(wb_hbm.at[pl.ds(crow0, NCHUNK)], wb_v)
    pltpu.sync_copy(wc_hbm.at[pl.ds(crow0, NCHUNK)], wc_v)

    slots = ((ra1, ra2, ra3, sem_a), (rb1, rb2, rb3, sem_b))

    def issue(g, slot):
        r1, r2, r3, sem = slot
        pltpu.async_copy(t_hbm.at[idx1_v.at[g]], r1, sem)
        pltpu.async_copy(t_hbm.at[idx2_v.at[g]], r2, sem)
        pltpu.async_copy(t_hbm.at[idx3_v.at[g]], r3, sem)

    def drain(g, slot):
        r1, r2, r3, sem = slot
        pltpu.make_async_copy(t_hbm.at[idx1_v.at[g]], r1, sem).wait()
        pltpu.make_async_copy(t_hbm.at[idx2_v.at[g]], r2, sem).wait()
        pltpu.make_async_copy(t_hbm.at[idx3_v.at[g]], r3, sem).wait()

    def drain_out():
        pltpu.make_async_copy(out_v, out_hbm.at[pl.ds(0, G)], sem_o).wait()

    dnums = lax.GatherDimensionNumbers(
        offset_dims=(), collapsed_slice_dims=(0,), start_index_map=(0,))

    def bcast(v16, lane):
        idx = jnp.full((16, 1), lane, jnp.int32)
        return lax.gather(v16, idx, dnums, (1,),
                          mode=lax.GatherScatterMode.PROMISE_IN_BOUNDS)

    def compute(g, slot):
        r1, r2, r3, _ = slot

        def row16(q, _):
            wa16 = wa_v[g, pl.ds(q * 16, 16)]
            wb16 = wb_v[g, pl.ds(q * 16, 16)]
            wc16 = wc_v[g, pl.ds(q * 16, 16)]
            for ri in range(16):
                r = q * 16 + ri
                wa = bcast(wa16, ri)
                wb = bcast(wb16, ri)
                wc = bcast(wc16, ri)
                for j in range(C2 // 16):
                    sl = pl.ds(j * 16, 16)
                    out_v[r, sl] = (wa * r1[r, sl] + wb * r2[r, sl]
                                    + wc * r3[r, sl])
            return 0

        lax.fori_loop(0, G // 16, row16, 0)

    issue(0, slots[0])

    def step(i, _):
        gg = 2 * i
        for b in (0, 1):
            g = gg + b
            slot = slots[b]
            other = slots[1 - b]
            drain(g, slot)

            @pl.when(g + 1 < NCHUNK)
            def _():
                issue(g + 1, other)

            @pl.when(g >= 1)
            def _():
                drain_out()

            compute(g, slot)
            pltpu.async_copy(out_v, out_hbm.at[pl.ds(row0 + g * G, G)], sem_o)
        return 0

    lax.fori_loop(0, NCHUNK // 2, step, 0)
    drain_out()


_interp_call = pl.kernel(
    _interp_sc,
    out_type=jax.ShapeDtypeStruct((NTOT, C2), jnp.float32),
    mesh=plsc.VectorSubcoreMesh(core_axis_name="c", subcore_axis_name="s"),
    scratch_types=[
        pltpu.VMEM((NCHUNK, G), jnp.int32),
        pltpu.VMEM((NCHUNK, G), jnp.int32),
        pltpu.VMEM((NCHUNK, G), jnp.int32),
        pltpu.VMEM((NCHUNK, G), jnp.float32),
        pltpu.VMEM((NCHUNK, G), jnp.float32),
        pltpu.VMEM((NCHUNK, G), jnp.float32),
        pltpu.VMEM((G, C2), jnp.float32),
        pltpu.VMEM((G, C2), jnp.float32),
        pltpu.VMEM((G, C2), jnp.float32),
        pltpu.VMEM((G, C2), jnp.float32),
        pltpu.VMEM((G, C2), jnp.float32),
        pltpu.VMEM((G, C2), jnp.float32),
        pltpu.VMEM((G, C2), jnp.float32),
        pltpu.SemaphoreType.DMA,
        pltpu.SemaphoreType.DMA,
        pltpu.SemaphoreType.DMA,
    ],
)


@jax.jit
def kernel(xyz1, xyz2, points1, points2, W1, b1, W2, b2):
    xyz1p = jnp.pad(xyz1, ((0, 0), (0, 0), (0, 5)))
    xyz2t = jnp.pad(xyz2, ((0, 0), (0, 0), (0, 5))).transpose(0, 2, 1)
    w1a = W1[:C2]
    w1b = W1[C2:]

    grid = (B, NB)
    shp_i = jax.ShapeDtypeStruct((B * NB, 1, BN), jnp.int32)
    shp_f = jax.ShapeDtypeStruct((B * NB, 1, BN), jnp.float32)
    out_spec = pl.BlockSpec((1, 1, BN), lambda b, n: (b * NB + n, 0, 0))
    i1, i2, i3, wa, wb, wc = pl.pallas_call(
        _knn_kernel,
        grid=grid,
        in_specs=[
            pl.BlockSpec((1, BN, 8), lambda b, n: (b, n, 0)),
            pl.BlockSpec((1, 8, N2), lambda b, n: (b, 0, 0)),
        ],
        out_specs=[out_spec] * 6,
        out_shape=[shp_i, shp_i, shp_i, shp_f, shp_f, shp_f],
    )(xyz1p, xyz2t)

    p2f = points2.reshape(B * N2, C2)
    table = pl.pallas_call(
        _table_kernel,
        grid=(B * N2 // 512,),
        in_specs=[
            pl.BlockSpec((512, C2), lambda i: (i, 0)),
            pl.BlockSpec((C2, C2), lambda i: (0, 0)),
        ],
        out_specs=pl.BlockSpec((512, C2), lambda i: (i, 0)),
        out_shape=jax.ShapeDtypeStruct((B * N2, C2), jnp.float32),
    )(p2f, w1a)

    nc = NW * NCHUNK
    interp = _interp_call(
        table,
        i1.reshape(nc, G), i2.reshape(nc, G), i3.reshape(nc, G),
        wa.reshape(nc, G), wb.reshape(nc, G), wc.reshape(nc, G))

    p1f = points1.reshape(NTOT, C1)
    out = pl.pallas_call(
        _mlp_kernel,
        grid=(NTOT // BN,),
        in_specs=[
            pl.BlockSpec((BN, C2), lambda i: (i, 0)),
            pl.BlockSpec((BN, C1), lambda i: (i, 0)),
            pl.BlockSpec((C1, C2), lambda i: (0, 0)),
            pl.BlockSpec((C2, C2), lambda i: (0, 0)),
            pl.BlockSpec((1, C2), lambda i: (0, 0)),
            pl.BlockSpec((1, C2), lambda i: (0, 0)),
        ],
        out_specs=pl.BlockSpec((BN, C2), lambda i: (i, 0)),
        out_shape=jax.ShapeDtypeStruct((NTOT, C2), jnp.float32),
    )(interp, p1f, w1b, W2, b1.reshape(1, C2), b2.reshape(1, C2))
    return out.reshape(B, N1, C2)


# speedup vs baseline: 1.2610x; 1.2055x over previous
import functools

import jax
import jax.numpy as jnp
from jax import lax
from jax.experimental import pallas as pl
from jax.experimental.pallas import tpu as pltpu
from jax.experimental.pallas import tpu_sc as plsc

BN = 256
N2 = 1024
C2 = 256
C1 = 128
BIG_I = 1 << 30
INF = 3e38

B = 8
N1 = 4096
NB = N1 // BN
NTOT = B * N1
NW = 32
G = 32
HB = B // 2
NH = HB * N1


def _knn_kernel(base0, xyz1_ref, xyz2t_ref, i1_ref, i2_ref, i3_ref,
                w1_ref, w2_ref, w3_ref):
    b = pl.program_id(0)
    x1 = xyz1_ref[0]
    x2t = xyz2t_ref[0]
    sq1 = jnp.sum(x1 * x1, axis=1, keepdims=True)
    sq2 = jnp.sum(x2t * x2t, axis=0, keepdims=True)
    dot = lax.dot_general(x1, x2t, (((1,), (0,)), ((), ())),
                          preferred_element_type=jnp.float32)
    d = sq1 + sq2 - 2.0 * dot

    cidx = lax.broadcasted_iota(jnp.int32, (BN, N2), 1)
    m1 = jnp.min(d, axis=1, keepdims=True)
    msk1 = d == m1
    i1 = jnp.min(jnp.where(msk1, cidx, BIG_I), axis=1, keepdims=True)
    e = jnp.where(msk1, INF, d)
    m2 = jnp.min(e, axis=1, keepdims=True)
    msk2 = e == m2
    i2 = jnp.min(jnp.where(msk2, cidx, BIG_I), axis=1, keepdims=True)
    f = jnp.where(msk2, INF, e)
    m3 = jnp.min(f, axis=1, keepdims=True)
    i3 = jnp.min(jnp.where(f == m3, cidx, BIG_I), axis=1, keepdims=True)

    r1 = 1.0 / jnp.maximum(m1, 1e-10)
    r2 = 1.0 / jnp.maximum(m2, 1e-10)
    r3 = 1.0 / jnp.maximum(m3, 1e-10)
    norm = r1 + r2 + r3
    base = base0 + b * N2
    i1_ref[0] = (i1 + base).reshape(1, BN)
    i2_ref[0] = (i2 + base).reshape(1, BN)
    i3_ref[0] = (i3 + base).reshape(1, BN)
    w1_ref[0] = (r1 / norm).reshape(1, BN)
    w2_ref[0] = (r2 / norm).reshape(1, BN)
    w3_ref[0] = (r3 / norm).reshape(1, BN)


def _table_kernel(p2_ref, w1a_ref, t_ref):
    t_ref[...] = jnp.dot(p2_ref[...], w1a_ref[...],
                         preferred_element_type=jnp.float32)


def _mlp_kernel(x_ref, p1_ref, w1b_ref, w2_ref, b1_ref, b2_ref, out_ref):
    h = x_ref[...] + p1_ref[...] @ w1b_ref[...] + b1_ref[...]
    h = jnp.maximum(h, 0.0)
    o = h @ w2_ref[...] + b2_ref[...]
    out_ref[...] = jnp.maximum(o, 0.0)


@functools.lru_cache(maxsize=None)
def _make_interp(nrows):
    rows_per_w = nrows // NW
    nchunk = rows_per_w // G

    def _interp_sc(t_hbm, i1_hbm, i2_hbm, i3_hbm, wa_hbm, wb_hbm, wc_hbm,
                   out_hbm, idx1_v, idx2_v, idx3_v, wa_v, wb_v, wc_v,
                   ra1, ra2, ra3, rb1, rb2, rb3, out_v,
                   sem_a, sem_b, sem_o):
        wid = lax.axis_index("s") * 2 + lax.axis_index("c")
        row0 = wid * rows_per_w
        crow0 = wid * nchunk

        pltpu.sync_copy(i1_hbm.at[pl.ds(crow0, nchunk)], idx1_v)
        pltpu.sync_copy(i2_hbm.at[pl.ds(crow0, nchunk)], idx2_v)
        pltpu.sync_copy(i3_hbm.at[pl.ds(crow0, nchunk)], idx3_v)
        pltpu.sync_copy(wa_hbm.at[pl.ds(crow0, nchunk)], wa_v)
        pltpu.sync_copy(wb_hbm.at[pl.ds(crow0, nchunk)], wb_v)
        pltpu.sync_copy(wc_hbm.at[pl.ds(crow0, nchunk)], wc_v)

        slots = ((ra1, ra2, ra3, sem_a), (rb1, rb2, rb3, sem_b))

        def issue(g, slot):
            r1, r2, r3, sem = slot
            pltpu.async_copy(t_hbm.at[idx1_v.at[g]], r1, sem)
            pltpu.async_copy(t_hbm.at[idx2_v.at[g]], r2, sem)
            pltpu.async_copy(t_hbm.at[idx3_v.at[g]], r3, sem)

        def drain(g, slot):
            r1, r2, r3, sem = slot
            pltpu.make_async_copy(t_hbm.at[idx1_v.at[g]], r1, sem).wait()
            pltpu.make_async_copy(t_hbm.at[idx2_v.at[g]], r2, sem).wait()
            pltpu.make_async_copy(t_hbm.at[idx3_v.at[g]], r3, sem).wait()

        def drain_out():
            pltpu.make_async_copy(out_v, out_hbm.at[pl.ds(0, G)], sem_o).wait()

        dnums = lax.GatherDimensionNumbers(
            offset_dims=(), collapsed_slice_dims=(0,), start_index_map=(0,))

        def bcast(v16, lane):
            idx = jnp.full((16, 1), lane, jnp.int32)
            return lax.gather(v16, idx, dnums, (1,),
                              mode=lax.GatherScatterMode.PROMISE_IN_BOUNDS)

        def compute(g, slot):
            r1, r2, r3, _ = slot

            def row16(q, _):
                wa16 = wa_v[g, pl.ds(q * 16, 16)]
                wb16 = wb_v[g, pl.ds(q * 16, 16)]
                wc16 = wc_v[g, pl.ds(q * 16, 16)]
                for ri in range(16):
                    r = q * 16 + ri
                    wa = bcast(wa16, ri)
                    wb = bcast(wb16, ri)
                    wc = bcast(wc16, ri)
                    for j in range(C2 // 16):
                        sl = pl.ds(j * 16, 16)
                        out_v[r, sl] = (wa * r1[r, sl] + wb * r2[r, sl]
                                        + wc * r3[r, sl])
                return 0

            lax.fori_loop(0, G // 16, row16, 0)

        issue(0, slots[0])

        def step(i, _):
            gg = 2 * i
            for bb in (0, 1):
                g = gg + bb
                slot = slots[bb]
                other = slots[1 - bb]
                drain(g, slot)

                @pl.when(g + 1 < nchunk)
                def _():
                    issue(g + 1, other)

                @pl.when(g >= 1)
                def _():
                    drain_out()

                compute(g, slot)
                pltpu.async_copy(out_v, out_hbm.at[pl.ds(row0 + g * G, G)],
                                 sem_o)
            return 0

        lax.fori_loop(0, nchunk // 2, step, 0)
        drain_out()

    return pl.kernel(
        _interp_sc,
        out_type=jax.ShapeDtypeStruct((nrows, C2), jnp.float32),
        mesh=plsc.VectorSubcoreMesh(core_axis_name="c", subcore_axis_name="s"),
        scratch_types=[
            pltpu.VMEM((nchunk, G), jnp.int32),
            pltpu.VMEM((nchunk, G), jnp.int32),
            pltpu.VMEM((nchunk, G), jnp.int32),
            pltpu.VMEM((nchunk, G), jnp.float32),
            pltpu.VMEM((nchunk, G), jnp.float32),
            pltpu.VMEM((nchunk, G), jnp.float32),
            pltpu.VMEM((G, C2), jnp.float32),
            pltpu.VMEM((G, C2), jnp.float32),
            pltpu.VMEM((G, C2), jnp.float32),
            pltpu.VMEM((G, C2), jnp.float32),
            pltpu.VMEM((G, C2), jnp.float32),
            pltpu.VMEM((G, C2), jnp.float32),
            pltpu.VMEM((G, C2), jnp.float32),
            pltpu.SemaphoreType.DMA,
            pltpu.SemaphoreType.DMA,
            pltpu.SemaphoreType.DMA,
        ],
    )


def _knn_half(xyz1p_h, xyz2t_h, half):
    shp_i = jax.ShapeDtypeStruct((HB * NB, 1, BN), jnp.int32)
    shp_f = jax.ShapeDtypeStruct((HB * NB, 1, BN), jnp.float32)
    out_spec = pl.BlockSpec((1, 1, BN), lambda b, n: (b * NB + n, 0, 0))
    return pl.pallas_call(
        functools.partial(_knn_kernel, half * HB * N2),
        grid=(HB, NB),
        in_specs=[
            pl.BlockSpec((1, BN, 8), lambda b, n: (b, n, 0)),
            pl.BlockSpec((1, 8, N2), lambda b, n: (b, 0, 0)),
        ],
        out_specs=[out_spec] * 6,
        out_shape=[shp_i, shp_i, shp_i, shp_f, shp_f, shp_f],
    )(xyz1p_h, xyz2t_h)


def _mlp_half(interp_h, p1_h, w1b, W2, b1r, b2r):
    return pl.pallas_call(
        _mlp_kernel,
        grid=(NH // BN,),
        in_specs=[
            pl.BlockSpec((BN, C2), lambda i: (i, 0)),
            pl.BlockSpec((BN, C1), lambda i: (i, 0)),
            pl.BlockSpec((C1, C2), lambda i: (0, 0)),
            pl.BlockSpec((C2, C2), lambda i: (0, 0)),
            pl.BlockSpec((1, C2), lambda i: (0, 0)),
            pl.BlockSpec((1, C2), lambda i: (0, 0)),
        ],
        out_specs=pl.BlockSpec((BN, C2), lambda i: (i, 0)),
        out_shape=jax.ShapeDtypeStruct((NH, C2), jnp.float32),
    )(interp_h, p1_h, w1b, W2, b1r, b2r)


@jax.jit
def kernel(xyz1, xyz2, points1, points2, W1, b1, W2, b2):
    xyz1p = jnp.pad(xyz1, ((0, 0), (0, 0), (0, 5)))
    xyz2t = jnp.pad(xyz2, ((0, 0), (0, 0), (0, 5))).transpose(0, 2, 1)
    w1a = W1[:C2]
    w1b = W1[C2:]
    b1r = b1.reshape(1, C2)
    b2r = b2.reshape(1, C2)

    p2f = points2.reshape(B * N2, C2)
    table = pl.pallas_call(
        _table_kernel,
        grid=(B * N2 // 512,),
        in_specs=[
            pl.BlockSpec((512, C2), lambda i: (i, 0)),
            pl.BlockSpec((C2, C2), lambda i: (0, 0)),
        ],
        out_specs=pl.BlockSpec((512, C2), lambda i: (i, 0)),
        out_shape=jax.ShapeDtypeStruct((B * N2, C2), jnp.float32),
    )(p2f, w1a)

    interp_call = _make_interp(NH)
    nc = NH // G
    p1f = points1.reshape(NTOT, C1)
    outs = []
    knn = [None, None]
    for h in (0, 1):
        sl = slice(h * HB, (h + 1) * HB)
        knn[h] = _knn_half(xyz1p[sl], xyz2t[sl], h)
    for h in (0, 1):
        i1, i2, i3, wa, wb, wc = knn[h]
        interp_h = interp_call(
            table,
            i1.reshape(nc, G), i2.reshape(nc, G), i3.reshape(nc, G),
            wa.reshape(nc, G), wb.reshape(nc, G), wc.reshape(nc, G))
        outs.append(_mlp_half(interp_h, p1f[h * NH:(h + 1) * NH],
                              w1b, W2, b1r, b2r))
    return jnp.concatenate(outs, axis=0).reshape(B, N1, C2)


# speedup vs baseline: 1.6552x; 1.3126x over previous
import functools

import jax
import jax.numpy as jnp
from jax import lax
from jax.experimental import pallas as pl
from jax.experimental.pallas import tpu as pltpu
from jax.experimental.pallas import tpu_sc as plsc

BN = 256
N2 = 1024
C2 = 256
C1 = 128
BIG_I = 1 << 30
INF = 3e38

B = 8
N1 = 4096
NB = N1 // BN
NTOT = B * N1
NW = 32
G = 32
HB = B // 2
NH = HB * N1


def _knn_kernel(base0, xyz1_ref, xyz2_ref, i1_ref, i2_ref, i3_ref,
                w1_ref, w2_ref, w3_ref):
    b = pl.program_id(0)
    x1 = xyz1_ref[0]
    x2 = xyz2_ref[0]
    sq2 = jnp.sum(x2 * x2, axis=1, keepdims=True)
    sq1r = lax.dot_general(jnp.ones((1, 3), jnp.float32), x1 * x1,
                           (((1,), (1,)), ((), ())),
                           precision=lax.Precision.HIGHEST,
                           preferred_element_type=jnp.float32)
    cross = lax.dot_general(x2, x1, (((1,), (1,)), ((), ())),
                            preferred_element_type=jnp.float32)
    dT = sq2 - 2.0 * cross + sq1r

    ridx = lax.broadcasted_iota(jnp.int32, (N2, BN), 0)
    m1 = jnp.min(dT, axis=0, keepdims=True)
    msk1 = dT == m1
    i1 = jnp.min(jnp.where(msk1, ridx, BIG_I), axis=0, keepdims=True)
    e = jnp.where(msk1, INF, dT)
    m2 = jnp.min(e, axis=0, keepdims=True)
    msk2 = e == m2
    i2 = jnp.min(jnp.where(msk2, ridx, BIG_I), axis=0, keepdims=True)
    f = jnp.where(msk2, INF, e)
    m3 = jnp.min(f, axis=0, keepdims=True)
    i3 = jnp.min(jnp.where(f == m3, ridx, BIG_I), axis=0, keepdims=True)

    r1 = 1.0 / jnp.maximum(m1, 1e-10)
    r2 = 1.0 / jnp.maximum(m2, 1e-10)
    r3 = 1.0 / jnp.maximum(m3, 1e-10)
    norm = r1 + r2 + r3
    base = base0 + b * N2
    i1_ref[0] = i1 + base
    i2_ref[0] = i2 + base
    i3_ref[0] = i3 + base
    w1_ref[0] = r1 / norm
    w2_ref[0] = r2 / norm
    w3_ref[0] = r3 / norm


def _table_kernel(p2_ref, w1a_ref, t_ref):
    t_ref[...] = jnp.dot(p2_ref[...], w1a_ref[...],
                         preferred_element_type=jnp.float32)


def _mlp_kernel(x_ref, p1_ref, w1b_ref, w2_ref, b1_ref, b2_ref, out_ref):
    h = x_ref[...] + p1_ref[...] @ w1b_ref[...] + b1_ref[...]
    h = jnp.maximum(h, 0.0)
    o = h @ w2_ref[...] + b2_ref[...]
    out_ref[...] = jnp.maximum(o, 0.0)


@functools.lru_cache(maxsize=None)
def _make_interp(nrows):
    rows_per_w = nrows // NW
    nchunk = rows_per_w // G

    def _interp_sc(t_hbm, i1_hbm, i2_hbm, i3_hbm, wa_hbm, wb_hbm, wc_hbm,
                   out_hbm, idx1_v, idx2_v, idx3_v, wa_v, wb_v, wc_v,
                   ra1, ra2, ra3, rb1, rb2, rb3, out_v,
                   sem_a, sem_b, sem_o):
        wid = lax.axis_index("s") * 2 + lax.axis_index("c")
        row0 = wid * rows_per_w
        crow0 = wid * nchunk

        pltpu.sync_copy(i1_hbm.at[pl.ds(crow0, nchunk)], idx1_v)
        pltpu.sync_copy(i2_hbm.at[pl.ds(crow0, nchunk)], idx2_v)
        pltpu.sync_copy(i3_hbm.at[pl.ds(crow0, nchunk)], idx3_v)
        pltpu.sync_copy(wa_hbm.at[pl.ds(crow0, nchunk)], wa_v)
        pltpu.sync_copy(wb_hbm.at[pl.ds(crow0, nchunk)], wb_v)
        pltpu.sync_copy(wc_hbm.at[pl.ds(crow0, nchunk)], wc_v)

        slots = ((ra1, ra2, ra3, sem_a), (rb1, rb2, rb3, sem_b))

        def issue(g, slot):
            r1, r2, r3, sem = slot
            pltpu.async_copy(t_hbm.at[idx1_v.at[g]], r1, sem)
            pltpu.async_copy(t_hbm.at[idx2_v.at[g]], r2, sem)
            pltpu.async_copy(t_hbm.at[idx3_v.at[g]], r3, sem)

        def drain(g, slot):
            r1, r2, r3, sem = slot
            pltpu.make_async_copy(t_hbm.at[idx1_v.at[g]], r1, sem).wait()
            pltpu.make_async_copy(t_hbm.at[idx2_v.at[g]], r2, sem).wait()
            pltpu.make_async_copy(t_hbm.at[idx3_v.at[g]], r3, sem).wait()

        def drain_out():
            pltpu.make_async_copy(out_v, out_hbm.at[pl.ds(0, G)], sem_o).wait()

        dnums = lax.GatherDimensionNumbers(
            offset_dims=(), collapsed_slice_dims=(0,), start_index_map=(0,))

        def bcast(v16, lane):
            idx = jnp.full((16, 1), lane, jnp.int32)
            return lax.gather(v16, idx, dnums, (1,),
                              mode=lax.GatherScatterMode.PROMISE_IN_BOUNDS)

        def compute(g, slot):
            r1, r2, r3, _ = slot

            def row16(q, _):
                wa16 = wa_v[g, pl.ds(q * 16, 16)]
                wb16 = wb_v[g, pl.ds(q * 16, 16)]
                wc16 = wc_v[g, pl.ds(q * 16, 16)]
                for ri in range(16):
                    r = q * 16 + ri
                    wa = bcast(wa16, ri)
                    wb = bcast(wb16, ri)
                    wc = bcast(wc16, ri)
                    for j in range(C2 // 16):
                        sl = pl.ds(j * 16, 16)
                        out_v[r, sl] = (wa * r1[r, sl] + wb * r2[r, sl]
                                        + wc * r3[r, sl])
                return 0

            lax.fori_loop(0, G // 16, row16, 0)

        issue(0, slots[0])

        def step(i, _):
            gg = 2 * i
            for bb in (0, 1):
                g = gg + bb
                slot = slots[bb]
                other = slots[1 - bb]
                drain(g, slot)

                @pl.when(g + 1 < nchunk)
                def _():
                    issue(g + 1, other)

                @pl.when(g >= 1)
                def _():
                    drain_out()

                compute(g, slot)
                pltpu.async_copy(out_v, out_hbm.at[pl.ds(row0 + g * G, G)],
                                 sem_o)
            return 0

        lax.fori_loop(0, nchunk // 2, step, 0)
        drain_out()

    return pl.kernel(
        _interp_sc,
        out_type=jax.ShapeDtypeStruct((nrows, C2), jnp.float32),
        mesh=plsc.VectorSubcoreMesh(core_axis_name="c", subcore_axis_name="s"),
        scratch_types=[
            pltpu.VMEM((nchunk, G), jnp.int32),
            pltpu.VMEM((nchunk, G), jnp.int32),
            pltpu.VMEM((nchunk, G), jnp.int32),
            pltpu.VMEM((nchunk, G), jnp.float32),
            pltpu.VMEM((nchunk, G), jnp.float32),
            pltpu.VMEM((nchunk, G), jnp.float32),
            pltpu.VMEM((G, C2), jnp.float32),
            pltpu.VMEM((G, C2), jnp.float32),
            pltpu.VMEM((G, C2), jnp.float32),
            pltpu.VMEM((G, C2), jnp.float32),
            pltpu.VMEM((G, C2), jnp.float32),
            pltpu.VMEM((G, C2), jnp.float32),
            pltpu.VMEM((G, C2), jnp.float32),
            pltpu.SemaphoreType.DMA,
            pltpu.SemaphoreType.DMA,
            pltpu.SemaphoreType.DMA,
        ],
    )


def _knn_half(xyz1_h, xyz2_h, half):
    shp_i = jax.ShapeDtypeStruct((HB * NB, 1, BN), jnp.int32)
    shp_f = jax.ShapeDtypeStruct((HB * NB, 1, BN), jnp.float32)
    out_spec = pl.BlockSpec((1, 1, BN), lambda b, n: (b * NB + n, 0, 0))
    return pl.pallas_call(
        functools.partial(_knn_kernel, half * HB * N2),
        grid=(HB, NB),
        in_specs=[
            pl.BlockSpec((1, BN, 3), lambda b, n: (b, n, 0)),
            pl.BlockSpec((1, N2, 3), lambda b, n: (b, 0, 0)),
        ],
        out_specs=[out_spec] * 6,
        out_shape=[shp_i, shp_i, shp_i, shp_f, shp_f, shp_f],
    )(xyz1_h, xyz2_h)


def _mlp_half(interp_h, p1_h, w1b, W2, b1r, b2r):
    return pl.pallas_call(
        _mlp_kernel,
        grid=(NH // BN,),
        in_specs=[
            pl.BlockSpec((BN, C2), lambda i: (i, 0)),
            pl.BlockSpec((BN, C1), lambda i: (i, 0)),
            pl.BlockSpec((C1, C2), lambda i: (0, 0)),
            pl.BlockSpec((C2, C2), lambda i: (0, 0)),
            pl.BlockSpec((1, C2), lambda i: (0, 0)),
            pl.BlockSpec((1, C2), lambda i: (0, 0)),
        ],
        out_specs=pl.BlockSpec((BN, C2), lambda i: (i, 0)),
        out_shape=jax.ShapeDtypeStruct((NH, C2), jnp.float32),
    )(interp_h, p1_h, w1b, W2, b1r, b2r)


@jax.jit
def kernel(xyz1, xyz2, points1, points2, W1, b1, W2, b2):
    w1a = W1[:C2]
    w1b = W1[C2:]
    b1r = b1.reshape(1, C2)
    b2r = b2.reshape(1, C2)

    p2f = points2.reshape(B * N2, C2)
    table = pl.pallas_call(
        _table_kernel,
        grid=(B * N2 // 512,),
        in_specs=[
            pl.BlockSpec((512, C2), lambda i: (i, 0)),
            pl.BlockSpec((C2, C2), lambda i: (0, 0)),
        ],
        out_specs=pl.BlockSpec((512, C2), lambda i: (i, 0)),
        out_shape=jax.ShapeDtypeStruct((B * N2, C2), jnp.float32),
    )(p2f, w1a)

    interp_call = _make_interp(NH)
    nc = NH // G
    p1f = points1.reshape(NTOT, C1)
    outs = []
    knn = [None, None]
    for h in (0, 1):
        sl = slice(h * HB, (h + 1) * HB)
        knn[h] = _knn_half(xyz1[sl], xyz2[sl], h)
    for h in (0, 1):
        i1, i2, i3, wa, wb, wc = knn[h]
        interp_h = interp_call(
            table,
            i1.reshape(nc, G), i2.reshape(nc, G), i3.reshape(nc, G),
            wa.reshape(nc, G), wb.reshape(nc, G), wc.reshape(nc, G))
        outs.append(_mlp_half(interp_h, p1f[h * NH:(h + 1) * NH],
                              w1b, W2, b1r, b2r))
    return jnp.concatenate(outs, axis=0).reshape(B, N1, C2)


# speedup vs baseline: 1.7532x; 1.0592x over previous
import functools

import jax
import jax.numpy as jnp
from jax import lax
from jax.experimental import pallas as pl
from jax.experimental.pallas import tpu as pltpu
from jax.experimental.pallas import tpu_sc as plsc

BN = 256
N2 = 1024
C2 = 256
C1 = 128
BIG_I = 1 << 30
INF = 3e38

B = 8
N1 = 4096
NB = N1 // BN
NTOT = B * N1
NW = 32
G = 32
HB = B // 2
NH = HB * N1


def _knn_kernel(base0, xyz1_ref, xyz2_ref, i1_ref, i2_ref, i3_ref,
                w1_ref, w2_ref, w3_ref):
    b = pl.program_id(0)
    x1 = xyz1_ref[0]
    x2 = xyz2_ref[0]
    sq2 = jnp.sum(x2 * x2, axis=1, keepdims=True)
    sq1r = lax.dot_general(jnp.ones((1, 3), jnp.float32), x1 * x1,
                           (((1,), (1,)), ((), ())),
                           precision=lax.Precision.HIGHEST,
                           preferred_element_type=jnp.float32)
    cross = lax.dot_general(x2, x1, (((1,), (1,)), ((), ())),
                            preferred_element_type=jnp.float32)
    dT = sq2 - 2.0 * cross + sq1r

    ridx = lax.broadcasted_iota(jnp.int32, (N2, BN), 0)
    m1 = jnp.min(dT, axis=0, keepdims=True)
    msk1 = dT == m1
    i1 = jnp.min(jnp.where(msk1, ridx, BIG_I), axis=0, keepdims=True)
    e = jnp.where(msk1, INF, dT)
    m2 = jnp.min(e, axis=0, keepdims=True)
    msk2 = e == m2
    i2 = jnp.min(jnp.where(msk2, ridx, BIG_I), axis=0, keepdims=True)
    f = jnp.where(msk2, INF, e)
    m3 = jnp.min(f, axis=0, keepdims=True)
    i3 = jnp.min(jnp.where(f == m3, ridx, BIG_I), axis=0, keepdims=True)

    r1 = 1.0 / jnp.maximum(m1, 1e-10)
    r2 = 1.0 / jnp.maximum(m2, 1e-10)
    r3 = 1.0 / jnp.maximum(m3, 1e-10)
    norm = r1 + r2 + r3
    base = base0 + b * N2
    i1_ref[0] = i1 + base
    i2_ref[0] = i2 + base
    i3_ref[0] = i3 + base
    w1_ref[0] = r1 / norm
    w2_ref[0] = r2 / norm
    w3_ref[0] = r3 / norm


def _table_kernel(p2_ref, w1a_ref, t_ref):
    t_ref[...] = jnp.dot(p2_ref[...], w1a_ref[...],
                         preferred_element_type=jnp.float32)


def _mlp_kernel(x_ref, p1_ref, w1b_ref, w2_ref, b1_ref, b2_ref, *rest):
    out_ref = rest[-1]
    h = x_ref[...] + p1_ref[...] @ w1b_ref[...] + b1_ref[...]
    h = jnp.maximum(h, 0.0)
    o = h @ w2_ref[...] + b2_ref[...]
    out_ref[...] = jnp.maximum(o, 0.0)


@functools.lru_cache(maxsize=None)
def _make_interp(nrows):
    rows_per_w = nrows // NW
    nchunk = rows_per_w // G

    def _interp_sc(t_hbm, i1_hbm, i2_hbm, i3_hbm, wa_hbm, wb_hbm, wc_hbm,
                   out_hbm, idx1_v, idx2_v, idx3_v, wa_v, wb_v, wc_v,
                   ra1, ra2, ra3, rb1, rb2, rb3, out_v,
                   sem_a, sem_b, sem_o):
        wid = lax.axis_index("s") * 2 + lax.axis_index("c")
        row0 = wid * rows_per_w
        crow0 = wid * nchunk

        pltpu.sync_copy(i1_hbm.at[pl.ds(crow0, nchunk)], idx1_v)
        pltpu.sync_copy(i2_hbm.at[pl.ds(crow0, nchunk)], idx2_v)
        pltpu.sync_copy(i3_hbm.at[pl.ds(crow0, nchunk)], idx3_v)
        pltpu.sync_copy(wa_hbm.at[pl.ds(crow0, nchunk)], wa_v)
        pltpu.sync_copy(wb_hbm.at[pl.ds(crow0, nchunk)], wb_v)
        pltpu.sync_copy(wc_hbm.at[pl.ds(crow0, nchunk)], wc_v)

        slots = ((ra1, ra2, ra3, sem_a), (rb1, rb2, rb3, sem_b))

        def issue(g, slot):
            r1, r2, r3, sem = slot
            pltpu.async_copy(t_hbm.at[idx1_v.at[g]], r1, sem)
            pltpu.async_copy(t_hbm.at[idx2_v.at[g]], r2, sem)
            pltpu.async_copy(t_hbm.at[idx3_v.at[g]], r3, sem)

        def drain(g, slot):
            r1, r2, r3, sem = slot
            pltpu.make_async_copy(t_hbm.at[idx1_v.at[g]], r1, sem).wait()
            pltpu.make_async_copy(t_hbm.at[idx2_v.at[g]], r2, sem).wait()
            pltpu.make_async_copy(t_hbm.at[idx3_v.at[g]], r3, sem).wait()

        def drain_out():
            pltpu.make_async_copy(out_v, out_hbm.at[pl.ds(0, G)], sem_o).wait()

        dnums = lax.GatherDimensionNumbers(
            offset_dims=(), collapsed_slice_dims=(0,), start_index_map=(0,))

        def bcast(v16, lane):
            idx = jnp.full((16, 1), lane, jnp.int32)
            return lax.gather(v16, idx, dnums, (1,),
                              mode=lax.GatherScatterMode.PROMISE_IN_BOUNDS)

        def compute(g, slot):
            r1, r2, r3, _ = slot

            def row16(q, _):
                wa16 = wa_v[g, pl.ds(q * 16, 16)]
                wb16 = wb_v[g, pl.ds(q * 16, 16)]
                wc16 = wc_v[g, pl.ds(q * 16, 16)]
                for ri in range(16):
                    r = q * 16 + ri
                    wa = bcast(wa16, ri)
                    wb = bcast(wb16, ri)
                    wc = bcast(wc16, ri)
                    for j in range(C2 // 16):
                        sl = pl.ds(j * 16, 16)
                        out_v[r, sl] = (wa * r1[r, sl] + wb * r2[r, sl]
                                        + wc * r3[r, sl])
                return 0

            lax.fori_loop(0, G // 16, row16, 0)

        issue(0, slots[0])

        def step(i, _):
            gg = 2 * i
            for bb in (0, 1):
                g = gg + bb
                slot = slots[bb]
                other = slots[1 - bb]
                drain(g, slot)

                @pl.when(g + 1 < nchunk)
                def _():
                    issue(g + 1, other)

                @pl.when(g >= 1)
                def _():
                    drain_out()

                compute(g, slot)
                pltpu.async_copy(out_v, out_hbm.at[pl.ds(row0 + g * G, G)],
                                 sem_o)
            return 0

        lax.fori_loop(0, nchunk // 2, step, 0)
        drain_out()

    return pl.kernel(
        _interp_sc,
        out_type=jax.ShapeDtypeStruct((nrows, C2), jnp.float32),
        mesh=plsc.VectorSubcoreMesh(core_axis_name="c", subcore_axis_name="s"),
        scratch_types=[
            pltpu.VMEM((nchunk, G), jnp.int32),
            pltpu.VMEM((nchunk, G), jnp.int32),
            pltpu.VMEM((nchunk, G), jnp.int32),
            pltpu.VMEM((nchunk, G), jnp.float32),
            pltpu.VMEM((nchunk, G), jnp.float32),
            pltpu.VMEM((nchunk, G), jnp.float32),
            pltpu.VMEM((G, C2), jnp.float32),
            pltpu.VMEM((G, C2), jnp.float32),
            pltpu.VMEM((G, C2), jnp.float32),
            pltpu.VMEM((G, C2), jnp.float32),
            pltpu.VMEM((G, C2), jnp.float32),
            pltpu.VMEM((G, C2), jnp.float32),
            pltpu.VMEM((G, C2), jnp.float32),
            pltpu.SemaphoreType.DMA,
            pltpu.SemaphoreType.DMA,
            pltpu.SemaphoreType.DMA,
        ],
    )


def _knn_half(xyz1_h, xyz2_h, half):
    shp_i = jax.ShapeDtypeStruct((HB * NB, 1, BN), jnp.int32)
    shp_f = jax.ShapeDtypeStruct((HB * NB, 1, BN), jnp.float32)
    out_spec = pl.BlockSpec((1, 1, BN), lambda b, n: (b * NB + n, 0, 0))
    return pl.pallas_call(
        functools.partial(_knn_kernel, half * HB * N2),
        grid=(HB, NB),
        in_specs=[
            pl.BlockSpec((1, BN, 3), lambda b, n: (b, n, 0)),
            pl.BlockSpec((1, N2, 3), lambda b, n: (b, 0, 0)),
        ],
        out_specs=[out_spec] * 6,
        out_shape=[shp_i, shp_i, shp_i, shp_f, shp_f, shp_f],
    )(xyz1_h, xyz2_h)


def _mlp_half(interp_h, p1_h, w1b, W2, b1r, b2r, half, dest=None):
    off = half * (NH // BN)
    in_specs = [
        pl.BlockSpec((BN, C2), lambda i: (i, 0)),
        pl.BlockSpec((BN, C1), lambda i: (i, 0)),
        pl.BlockSpec((C1, C2), lambda i: (0, 0)),
        pl.BlockSpec((C2, C2), lambda i: (0, 0)),
        pl.BlockSpec((1, C2), lambda i: (0, 0)),
        pl.BlockSpec((1, C2), lambda i: (0, 0)),
    ]
    args = [interp_h, p1_h, w1b, W2, b1r, b2r]
    kwargs = {}
    if dest is not None:
        in_specs.append(pl.BlockSpec(memory_space=pl.ANY))
        args.append(dest)
        kwargs["input_output_aliases"] = {6: 0}
    return pl.pallas_call(
        _mlp_kernel,
        grid=(NH // BN,),
        in_specs=in_specs,
        out_specs=pl.BlockSpec((BN, C2), lambda i: (i + off, 0)),
        out_shape=jax.ShapeDtypeStruct((NTOT, C2), jnp.float32),
        **kwargs,
    )(*args)


@jax.jit
def kernel(xyz1, xyz2, points1, points2, W1, b1, W2, b2):
    w1a = W1[:C2]
    w1b = W1[C2:]
    b1r = b1.reshape(1, C2)
    b2r = b2.reshape(1, C2)

    p2f = points2.reshape(B * N2, C2)
    table = pl.pallas_call(
        _table_kernel,
        grid=(B * N2 // 512,),
        in_specs=[
            pl.BlockSpec((512, C2), lambda i: (i, 0)),
            pl.BlockSpec((C2, C2), lambda i: (0, 0)),
        ],
        out_specs=pl.BlockSpec((512, C2), lambda i: (i, 0)),
        out_shape=jax.ShapeDtypeStruct((B * N2, C2), jnp.float32),
    )(p2f, w1a)

    interp_call = _make_interp(NH)
    nc = NH // G
    p1f = points1.reshape(NTOT, C1)
    knn = [None, None]
    for h in (0, 1):
        sl = slice(h * HB, (h + 1) * HB)
        knn[h] = _knn_half(xyz1[sl], xyz2[sl], h)
    dest = None
    for h in (0, 1):
        i1, i2, i3, wa, wb, wc = knn[h]
        interp_h = interp_call(
            table,
            i1.reshape(nc, G), i2.reshape(nc, G), i3.reshape(nc, G),
            wa.reshape(nc, G), wb.reshape(nc, G), wc.reshape(nc, G))
        dest = _mlp_half(interp_h, p1f[h * NH:(h + 1) * NH],
                         w1b, W2, b1r, b2r, h, dest)
    return dest.reshape(B, N1, C2)
